# baseline (speedup 1.0000x reference)
import functools

import jax
import jax.numpy as jnp
from jax import lax
from jax.experimental import pallas as pl
from jax.experimental.pallas import tpu as pltpu
from jax.experimental.pallas import tpu_sc as plsc

N = 10000
E = 320000
D = 128
H = 8
C = 128
HC = H * C
D_INNER = 512
EPS = 1e-5

NC = 2
NS = 16
NW = NC * NS

EDGES = E + N
EP = 330240
SPAN = EP // NW
B = 16
ITERS = SPAN // B
BB = 16
SPAN_B = EP // NS
ITERS_B = SPAN_B // BB
NP = 10240
ZR = NP // NS
PR = NP // 8
PB = NP // 2
ZO = PB // NS

f32 = jnp.float32
i32 = jnp.int32


def _pre_body(x_ref, wl_ref, bl_ref, wr_ref, br_ref, xl_ref, xr_ref):
    x = x_ref[...]
    xl_ref[...] = jnp.dot(x, wl_ref[...], preferred_element_type=f32) + bl_ref[...]
    xr_ref[...] = jnp.dot(x, wr_ref[...], preferred_element_type=f32) + br_ref[...]


def _pre(x, W_l, b_l, W_r, b_r):
    blk = 1000
    grid = (N // blk,)
    return pl.pallas_call(
        _pre_body,
        grid=grid,
        in_specs=[
            pl.BlockSpec((blk, D), lambda i: (i, 0)),
            pl.BlockSpec((D, HC), lambda i: (0, 0)),
            pl.BlockSpec((1, HC), lambda i: (0, 0)),
            pl.BlockSpec((D, HC), lambda i: (0, 0)),
            pl.BlockSpec((1, HC), lambda i: (0, 0)),
        ],
        out_specs=[
            pl.BlockSpec((blk, HC), lambda i: (i, 0)),
            pl.BlockSpec((blk, HC), lambda i: (i, 0)),
        ],
        out_shape=[
            jax.ShapeDtypeStruct((N, HC), f32),
            jax.ShapeDtypeStruct((N, HC), f32),
        ],
    )(x, W_l, b_l.reshape(1, HC), W_r, b_r.reshape(1, HC))


def _passA_body(src_hbm, dst_hbm, row_hbm, xl_hbm, xr_hbm, att_hbm,
                ex_hbm, denom_hbm,
                idx_s, idx_d, rowidx, rows_l, rows_r, exd_v, exs_v, att_v,
                zbuf, denom_sh, sem, semg1, semg2):
    c = lax.axis_index("c")
    s = lax.axis_index("s")
    wid = c * NS + s
    lane = lax.iota(i32, 16)
    zrow = jnp.zeros((16,), f32)

    for k in range(8 * (C // 16)):
        zbuf[k // 8, pl.ds((k % 8) * 16, 16)] = zrow

    @pl.when(s < 2)
    def _():
        def zcopy(r, _):
            pltpu.sync_copy(zbuf, denom_sh.at[pl.ds(s * (PR // 2) + r * 8, 8)])
            return 0
        lax.fori_loop(0, PR // 16, zcopy, 0)

    pltpu.sync_copy(att_hbm, att_v)
    plsc.subcore_barrier()

    base = wid * SPAN

    def step(it, _):
        off = base + it * B
        m1 = pltpu.async_copy(src_hbm.at[pl.ds(off, B)], idx_s, sem)
        m2 = pltpu.async_copy(dst_hbm.at[pl.ds(off, B)], idx_d, sem)
        m3 = pltpu.async_copy(row_hbm.at[pl.ds(off, B)], rowidx, sem)
        m1.wait(); m2.wait(); m3.wait()
        g1 = pltpu.async_copy(xl_hbm.at[idx_s], rows_l, semg1)
        g2 = pltpu.async_copy(xr_hbm.at[idx_d], rows_r, semg2)
        g1.wait(); g2.wait()

        def edge(e, _):
            svec = jnp.full((16,), -1e30, f32)
            for h in range(H):
                acc = jnp.zeros((16,), f32)
                for k in range(C // 16):
                    o = h * C + k * 16
                    v = rows_l[e, pl.ds(o, 16)] + rows_r[e, pl.ds(o, 16)]
                    lr = jnp.maximum(v, v * 0.2)
                    acc = acc + lr * att_v[h, pl.ds(k * 16, 16)]
                for shf in (8, 4, 2, 1):
                    acc = acc + acc.at[lane ^ shf].get(mode="promise_in_bounds")
                svec = jnp.where(lane == h, acc, svec)
            ex = jnp.exp(svec)
            exs_v[e, :] = ex
            for k in range(C // 16):
                exd_v[e, pl.ds(k * 16, 16)] = zrow
            ebc = jnp.zeros((16,), i32) + e
            dbc = plsc.load_gather(idx_d, [ebc])
            col = (dbc & 7) * 16 + lane
            plsc.store_scatter(exd_v, [ebc, col], ex)
            return 0

        lax.fori_loop(0, B, edge, 0)
        pltpu.sync_copy(exd_v, denom_sh.at[rowidx], add=True)
        pltpu.sync_copy(exs_v, ex_hbm.at[pl.ds(off, B)])
        return 0

    lax.fori_loop(0, ITERS, step, 0)
    plsc.subcore_barrier()

    @pl.when(s < 2)
    def _():
        pltpu.sync_copy(denom_sh.at[pl.ds(s * (PR // 2), PR // 2)],
                        denom_hbm.at[c, pl.ds(s * (PR // 2), PR // 2)])


def _passA(src, dst, row, xl, xr, att):
    mesh = plsc.VectorSubcoreMesh(core_axis_name="c", subcore_axis_name="s")
    kern = pl.kernel(
        _passA_body,
        compiler_params=pltpu.CompilerParams(needs_layout_passes=False),
        out_type=[
            jax.ShapeDtypeStruct((EP, 16), f32),
            jax.ShapeDtypeStruct((NC, PR, C), f32),
        ],
        mesh=mesh,
        scratch_types=[
            pltpu.VMEM((B,), i32),
            pltpu.VMEM((B,), i32),
            pltpu.VMEM((B,), i32),
            pltpu.VMEM((B, HC), f32),
            pltpu.VMEM((B, HC), f32),
            pltpu.VMEM((B, C), f32),
            pltpu.VMEM((B, 16), f32),
            pltpu.VMEM((H, C), f32),
            pltpu.VMEM((8, C), f32),
            pltpu.VMEM_SHARED((PR, C), f32),
            pltpu.SemaphoreType.DMA,
            pltpu.SemaphoreType.DMA,
            pltpu.SemaphoreType.DMA,
        ],
    )
    return kern(src, dst, row, xl, xr, att)


def _passB_body(src2_hbm, dst_hbm, row_hbm, orow_hbm, ex_hbm, xlh_hbm,
                d0_hbm, d1_hbm,
                out_hbm,
                idx_s, idx_d, rowidx, orowidx, rows_l, ex_v, d0_v, d1_v,
                contrib, zbuf, out_sh, sem, semg1, semg2, semg3):
    c = lax.axis_index("c")
    s = lax.axis_index("s")
    wid = c * NS + s
    lane = lax.iota(i32, 16)

    zrow = jnp.zeros((16,), f32)
    for k in range(8 * (C // 16)):
        zbuf[k // 8, pl.ds((k % 8) * 16, 16)] = zrow

    def zcopy(r, _):
        pltpu.sync_copy(zbuf, out_sh.at[pl.ds(s * ZO + r * 8, 8)])
        return 0
    lax.fori_loop(0, ZO // 8, zcopy, 0)
    plsc.subcore_barrier()

    base = s * SPAN_B

    def step(it, _):
        off = base + it * BB
        m1 = pltpu.async_copy(src2_hbm.at[c, pl.ds(off, BB)], idx_s, sem)
        m2 = pltpu.async_copy(dst_hbm.at[pl.ds(off, BB)], idx_d, sem)
        m3 = pltpu.async_copy(row_hbm.at[pl.ds(off, BB)], rowidx, sem)
        m5 = pltpu.async_copy(orow_hbm.at[pl.ds(off, BB)], orowidx, sem)
        m4 = pltpu.async_copy(ex_hbm.at[pl.ds(off, BB)], ex_v, sem)
        m1.wait(); m2.wait(); m3.wait(); m4.wait(); m5.wait()
        g1 = pltpu.async_copy(xlh_hbm.at[idx_s], rows_l, semg1)
        g2 = pltpu.async_copy(d0_hbm.at[rowidx], d0_v, semg2)
        g3 = pltpu.async_copy(d1_hbm.at[rowidx], d1_v, semg3)
        g1.wait(); g2.wait(); g3.wait()

        def edge(e, _):
            ebc = jnp.zeros((16,), i32) + e
            dbc = plsc.load_gather(idx_d, [ebc])
            col = (dbc & 7) * 16 + lane
            d = (plsc.load_gather(d0_v, [ebc, col])
                 + plsc.load_gather(d1_v, [ebc, col]))
            alpha = ex_v[e, :] / (d + 1e-16) * (1.0 / H)
            ah = [alpha.at[jnp.full((16,), h, i32)].get(mode="promise_in_bounds")
                  for h in range(H)]
            cb = (dbc & 1) * 64 + lane
            ncb = (1 - (dbc & 1)) * 64 + lane
            for k in range(4):
                acc = ah[0] * rows_l[e, pl.ds(k * 16, 16)]
                for h in range(1, H):
                    acc = acc + ah[h] * rows_l[e, pl.ds(h * 64 + k * 16, 16)]
                plsc.store_scatter(contrib, [ebc, cb + k * 16], acc)
                plsc.store_scatter(contrib, [ebc, ncb + k * 16], zrow)
            return 0

        lax.fori_loop(0, BB, edge, 0)
        pltpu.sync_copy(contrib, out_sh.at[orowidx], add=True)
        return 0

    lax.fori_loop(0, ITERS_B, step, 0)
    plsc.subcore_barrier()
    pltpu.sync_copy(out_sh.at[pl.ds(s * ZO, ZO)],
                    out_hbm.at[c, pl.ds(s * ZO, ZO)])


def _passB(src2, dst, row, orow, ex, xlh, d0, d1):
    mesh = plsc.VectorSubcoreMesh(core_axis_name="c", subcore_axis_name="s")
    kern = pl.kernel(
        _passB_body,
        compiler_params=pltpu.CompilerParams(needs_layout_passes=False),
        out_type=jax.ShapeDtypeStruct((NC, PB, C), f32),
        mesh=mesh,
        scratch_types=[
            pltpu.VMEM((BB,), i32),
            pltpu.VMEM((BB,), i32),
            pltpu.VMEM((BB,), i32),
            pltpu.VMEM((BB,), i32),
            pltpu.VMEM((BB, HC // 2), f32),
            pltpu.VMEM((BB, 16), f32),
            pltpu.VMEM((BB, C), f32),
            pltpu.VMEM((BB, C), f32),
            pltpu.VMEM((BB, C), f32),
            pltpu.VMEM((8, C), f32),
            pltpu.VMEM_SHARED((PB, C), f32),
            pltpu.SemaphoreType.DMA,
            pltpu.SemaphoreType.DMA,
            pltpu.SemaphoreType.DMA,
            pltpu.SemaphoreType.DMA,
        ],
    )
    return kern(src2, dst, row, orow, ex, xlh, d0, d1)


def _post_body(x_ref, olo_ref, ohi_ref, p_ref, w1_ref, b1_ref, w2_ref, out_ref):
    gat_bias = p_ref[0:1, :]
    g1, be1, m1, v1 = p_ref[1:2, :], p_ref[2:3, :], p_ref[3:4, :], p_ref[4:5, :]
    b2 = p_ref[5:6, :]
    g2, be2, m2, v2 = p_ref[6:7, :], p_ref[7:8, :], p_ref[8:9, :], p_ref[9:10, :]
    o = jnp.concatenate([olo_ref[...], ohi_ref[...]], axis=1)
    y = x_ref[...] + o + gat_bias
    y = (y - m1) * lax.rsqrt(v1 + EPS) * g1 + be1
    hid = jnp.maximum(jnp.dot(y, w1_ref[...], preferred_element_type=f32)
                      + b1_ref[...], 0.0)
    ff = jnp.dot(hid, w2_ref[...], preferred_element_type=f32) + b2
    z = y + ff
    out_ref[...] = (z - m2) * lax.rsqrt(v2 + EPS) * g2 + be2


def _post(x, olo, ohi, params, W1, b1, W2):
    blk = 1000
    grid = (N // blk,)
    return pl.pallas_call(
        _post_body,
        grid=grid,
        in_specs=[
            pl.BlockSpec((blk, D), lambda i: (i, 0)),
            pl.BlockSpec((blk, D // 2), lambda i: (i, 0)),
            pl.BlockSpec((blk, D // 2), lambda i: (i, 0)),
            pl.BlockSpec((10, D), lambda i: (0, 0)),
            pl.BlockSpec((D, D_INNER), lambda i: (0, 0)),
            pl.BlockSpec((1, D_INNER), lambda i: (0, 0)),
            pl.BlockSpec((D_INNER, D), lambda i: (0, 0)),
        ],
        out_specs=pl.BlockSpec((blk, D), lambda i: (i, 0)),
        out_shape=jax.ShapeDtypeStruct((N, D), f32),
    )(x, olo, ohi, params, W1, b1.reshape(1, D_INNER), W2)


def kernel(x, edge_index, W_l, b_l, W_r, b_r, att, gat_bias,
           bn1_gamma, bn1_beta, bn1_mean, bn1_var,
           W1, b1, W2, b2,
           bn2_gamma, bn2_beta, bn2_mean, bn2_var):
    loop = jnp.arange(N, dtype=i32)
    pad = EP - EDGES
    src = jnp.concatenate([edge_index[0].astype(i32), loop,
                           jnp.zeros((pad,), i32)])
    dst = jnp.concatenate([edge_index[1].astype(i32), loop,
                           jnp.full((pad,), N, i32)])

    row = lax.shift_right_logical(dst, 3)
    orow = lax.shift_right_logical(dst, 1)
    src2 = jnp.stack([src, src + N])

    xl, xr = _pre(x, W_l, b_l, W_r, b_r)
    ex, denom = _passA(src, dst, row, xl, xr, att)
    d0, d1 = denom[0], denom[1]
    xl3 = xl.reshape(N, H, C)
    xlh = jnp.concatenate([xl3[:, :, :C // 2].reshape(N, HC // 2),
                           xl3[:, :, C // 2:].reshape(N, HC // 2)], axis=0)
    out2 = _passB(src2, dst, row, orow, ex, xlh, d0, d1)
    olo = out2[0].reshape(NP, C // 2)[:N]
    ohi = out2[1].reshape(NP, C // 2)[:N]

    params = jnp.stack([gat_bias, bn1_gamma, bn1_beta, bn1_mean, bn1_var,
                        b2, bn2_gamma, bn2_beta, bn2_mean, bn2_var])
    return _post(x, olo, ohi, params, W1, b1, W2)


# speedup vs baseline: 1.5542x; 1.5542x over previous
import functools

import jax
import jax.numpy as jnp
from jax import lax
from jax.experimental import pallas as pl
from jax.experimental.pallas import tpu as pltpu
from jax.experimental.pallas import tpu_sc as plsc

N = 10000
E = 320000
D = 128
H = 8
C = 128
HC = H * C
D_INNER = 512
EPS = 1e-5

NC = 2
NS = 16
NW = NC * NS

EDGES = E + N
EP = 330240
SPAN = EP // NW
B = 40
ITERS = SPAN // B
BB = 40
SPAN_B = EP // NS
ITERS_B = SPAN_B // BB
NP = 10240
ZR = NP // NS
PR = NP // 8
PB = NP // 2
ZO = PB // NS

f32 = jnp.float32
i32 = jnp.int32


def _pre_body(x_ref, wl_ref, bl_ref, wr_ref, br_ref, xl_ref, xr_ref):
    x = x_ref[...]
    xl_ref[...] = jnp.dot(x, wl_ref[...], preferred_element_type=f32) + bl_ref[...]
    xr_ref[...] = jnp.dot(x, wr_ref[...], preferred_element_type=f32) + br_ref[...]


def _pre(x, W_l, b_l, W_r, b_r):
    blk = 1000
    grid = (N // blk,)
    return pl.pallas_call(
        _pre_body,
        grid=grid,
        in_specs=[
            pl.BlockSpec((blk, D), lambda i: (i, 0)),
            pl.BlockSpec((D, HC), lambda i: (0, 0)),
            pl.BlockSpec((1, HC), lambda i: (0, 0)),
            pl.BlockSpec((D, HC), lambda i: (0, 0)),
            pl.BlockSpec((1, HC), lambda i: (0, 0)),
        ],
        out_specs=[
            pl.BlockSpec((blk, HC), lambda i: (i, 0)),
            pl.BlockSpec((blk, HC), lambda i: (i, 0)),
        ],
        out_shape=[
            jax.ShapeDtypeStruct((N, HC), f32),
            jax.ShapeDtypeStruct((N, HC), f32),
        ],
    )(x, W_l, b_l.reshape(1, HC), W_r, b_r.reshape(1, HC))


def _passA_body(src_hbm, dst_hbm, row_hbm, xl_hbm, xr_hbm, att_hbm,
                ex_hbm, denom_hbm,
                idx_s, idx_d, rowidx, rows_l, rows_r, exd_v, exs_v, att_v,
                zbuf, denom_sh, sem, semg1, semg2):
    c = lax.axis_index("c")
    s = lax.axis_index("s")
    wid = c * NS + s
    lane = lax.iota(i32, 16)
    zrow = jnp.zeros((16,), f32)

    for k in range(8 * (C // 16)):
        zbuf[k // 8, pl.ds((k % 8) * 16, 16)] = zrow

    @pl.when(s < 2)
    def _():
        def zcopy(r, _):
            pltpu.sync_copy(zbuf, denom_sh.at[pl.ds(s * (PR // 2) + r * 8, 8)])
            return 0
        lax.fori_loop(0, PR // 16, zcopy, 0)

    pltpu.sync_copy(att_hbm, att_v)
    plsc.subcore_barrier()

    base = wid * SPAN

    def step(it, _):
        off = base + it * B
        m1 = pltpu.async_copy(src_hbm.at[pl.ds(off, B)], idx_s, sem)
        m2 = pltpu.async_copy(dst_hbm.at[pl.ds(off, B)], idx_d, sem)
        m3 = pltpu.async_copy(row_hbm.at[pl.ds(off, B)], rowidx, sem)
        m1.wait(); m2.wait(); m3.wait()
        g1 = pltpu.async_copy(xl_hbm.at[idx_s], rows_l, semg1)
        g2 = pltpu.async_copy(xr_hbm.at[idx_d], rows_r, semg2)
        g1.wait(); g2.wait()

        hmask = jnp.full((16,), -0x10000, i32)

        def edge(e, _):
            svec = jnp.full((16,), -1e30, f32)
            for h in range(H):
                acc = jnp.zeros((16,), f32)
                for g in range(C // 32):
                    o = h * (C // 2) + g * 16
                    ul = rows_l[e, pl.ds(o, 16)]
                    ur = rows_r[e, pl.ds(o, 16)]
                    lo = (plsc.bitcast(lax.shift_left(ul, 16), f32)
                          + plsc.bitcast(lax.shift_left(ur, 16), f32))
                    hi = (plsc.bitcast(ul & hmask, f32)
                          + plsc.bitcast(ur & hmask, f32))
                    llo = jnp.maximum(lo, lo * 0.2)
                    lhi = jnp.maximum(hi, hi * 0.2)
                    acc = acc + llo * att_v[h, pl.ds(g * 32, 16)]
                    acc = acc + lhi * att_v[h, pl.ds(g * 32 + 16, 16)]
                for shf in (8, 4, 2, 1):
                    acc = acc + acc.at[lane ^ shf].get(mode="promise_in_bounds")
                svec = jnp.where(lane == h, acc, svec)
            ex = jnp.exp(svec)
            exs_v[e, :] = ex
            for k in range(C // 16):
                exd_v[e, pl.ds(k * 16, 16)] = zrow
            ebc = jnp.zeros((16,), i32) + e
            dbc = plsc.load_gather(idx_d, [ebc])
            col = (dbc & 7) * 16 + lane
            plsc.store_scatter(exd_v, [ebc, col], ex)
            return 0

        lax.fori_loop(0, B, edge, 0)
        pltpu.sync_copy(exd_v, denom_sh.at[rowidx], add=True)
        pltpu.sync_copy(exs_v, ex_hbm.at[pl.ds(off, B)])
        return 0

    lax.fori_loop(0, ITERS, step, 0)
    plsc.subcore_barrier()

    @pl.when(s < 2)
    def _():
        pltpu.sync_copy(denom_sh.at[pl.ds(s * (PR // 2), PR // 2)],
                        denom_hbm.at[c, pl.ds(s * (PR // 2), PR // 2)])


def _passA(src, dst, row, xl, xr, att):
    mesh = plsc.VectorSubcoreMesh(core_axis_name="c", subcore_axis_name="s")
    kern = pl.kernel(
        _passA_body,
        compiler_params=pltpu.CompilerParams(needs_layout_passes=False),
        out_type=[
            jax.ShapeDtypeStruct((EP, 16), f32),
            jax.ShapeDtypeStruct((NC, PR, C), f32),
        ],
        mesh=mesh,
        scratch_types=[
            pltpu.VMEM((B,), i32),
            pltpu.VMEM((B,), i32),
            pltpu.VMEM((B,), i32),
            pltpu.VMEM((B, HC // 2), i32),
            pltpu.VMEM((B, HC // 2), i32),
            pltpu.VMEM((B, C), f32),
            pltpu.VMEM((B, 16), f32),
            pltpu.VMEM((H, C), f32),
            pltpu.VMEM((8, C), f32),
            pltpu.VMEM_SHARED((PR, C), f32),
            pltpu.SemaphoreType.DMA,
            pltpu.SemaphoreType.DMA,
            pltpu.SemaphoreType.DMA,
        ],
    )
    return kern(src, dst, row, xl, xr, att)


def _passB_body(src2_hbm, dst_hbm, row_hbm, orow_hbm, ex_hbm, xlh_hbm,
                d0_hbm, d1_hbm,
                out_hbm,
                idx_s, idx_d, rowidx, orowidx, rows_l, ex_v, d0_v, d1_v,
                contrib, zbuf, out_sh, sem, semg1, semg2, semg3):
    c = lax.axis_index("c")
    s = lax.axis_index("s")
    wid = c * NS + s
    lane = lax.iota(i32, 16)

    zrow = jnp.zeros((16,), f32)
    for k in range(8 * (C // 16)):
        zbuf[k // 8, pl.ds((k % 8) * 16, 16)] = zrow

    def zcopy(r, _):
        pltpu.sync_copy(zbuf, out_sh.at[pl.ds(s * ZO + r * 8, 8)])
        return 0
    lax.fori_loop(0, ZO // 8, zcopy, 0)
    plsc.subcore_barrier()

    base = s * SPAN_B

    def step(it, _):
        off = base + it * BB
        m1 = pltpu.async_copy(src2_hbm.at[pl.ds(c * EP + off, BB)], idx_s, sem)
        m2 = pltpu.async_copy(dst_hbm.at[pl.ds(off, BB)], idx_d, sem)
        m3 = pltpu.async_copy(row_hbm.at[pl.ds(off, BB)], rowidx, sem)
        m5 = pltpu.async_copy(orow_hbm.at[pl.ds(off, BB)], orowidx, sem)
        m4 = pltpu.async_copy(ex_hbm.at[pl.ds(off, BB)], ex_v, sem)
        m1.wait(); m2.wait(); m3.wait(); m4.wait(); m5.wait()
        g1 = pltpu.async_copy(xlh_hbm.at[idx_s], rows_l, semg1)
        g2 = pltpu.async_copy(d0_hbm.at[rowidx], d0_v, semg2)
        g3 = pltpu.async_copy(d1_hbm.at[rowidx], d1_v, semg3)
        g1.wait(); g2.wait(); g3.wait()

        def edge(e, _):
            ebc = jnp.zeros((16,), i32) + e
            dbc = plsc.load_gather(idx_d, [ebc])
            col = (dbc & 7) * 16 + lane
            d = (plsc.load_gather(d0_v, [ebc, col])
                 + plsc.load_gather(d1_v, [ebc, col]))
            alpha = ex_v[e, :] / (d + 1e-16) * (1.0 / H)
            ah = [alpha.at[jnp.full((16,), h, i32)].get(mode="promise_in_bounds")
                  for h in range(H)]
            cb = (dbc & 1) * 64 + lane
            ncb = (1 - (dbc & 1)) * 64 + lane
            hmask = jnp.full((16,), -0x10000, i32)
            for g in range(2):
                acc_lo = jnp.zeros((16,), f32)
                acc_hi = jnp.zeros((16,), f32)
                for h in range(H):
                    u = rows_l[e, pl.ds(h * 32 + g * 16, 16)]
                    acc_lo = acc_lo + ah[h] * plsc.bitcast(
                        lax.shift_left(u, 16), f32)
                    acc_hi = acc_hi + ah[h] * plsc.bitcast(u & hmask, f32)
                plsc.store_scatter(contrib, [ebc, cb + g * 32], acc_lo)
                plsc.store_scatter(contrib, [ebc, cb + g * 32 + 16], acc_hi)
                plsc.store_scatter(contrib, [ebc, ncb + g * 32], zrow)
                plsc.store_scatter(contrib, [ebc, ncb + g * 32 + 16], zrow)
            return 0

        lax.fori_loop(0, BB, edge, 0)
        pltpu.sync_copy(contrib, out_sh.at[orowidx], add=True)
        return 0

    lax.fori_loop(0, ITERS_B, step, 0)
    plsc.subcore_barrier()
    pltpu.sync_copy(out_sh.at[pl.ds(s * ZO, ZO)],
                    out_hbm.at[c, pl.ds(s * ZO, ZO)])


def _passB(src2, dst, row, orow, ex, xlh, d0, d1):
    mesh = plsc.VectorSubcoreMesh(core_axis_name="c", subcore_axis_name="s")
    kern = pl.kernel(
        _passB_body,
        compiler_params=pltpu.CompilerParams(needs_layout_passes=False),
        out_type=jax.ShapeDtypeStruct((NC, PB, C), f32),
        mesh=mesh,
        scratch_types=[
            pltpu.VMEM((BB,), i32),
            pltpu.VMEM((BB,), i32),
            pltpu.VMEM((BB,), i32),
            pltpu.VMEM((BB,), i32),
            pltpu.VMEM((BB, HC // 4), i32),
            pltpu.VMEM((BB, 16), f32),
            pltpu.VMEM((BB, C), f32),
            pltpu.VMEM((BB, C), f32),
            pltpu.VMEM((BB, C), f32),
            pltpu.VMEM((8, C), f32),
            pltpu.VMEM_SHARED((PB, C), f32),
            pltpu.SemaphoreType.DMA,
            pltpu.SemaphoreType.DMA,
            pltpu.SemaphoreType.DMA,
            pltpu.SemaphoreType.DMA,
        ],
    )
    return kern(src2, dst, row, orow, ex, xlh, d0, d1)


def _post_body(x_ref, olo_ref, ohi_ref, p_ref, w1_ref, b1_ref, w2_ref, out_ref):
    gat_bias = p_ref[0:1, :]
    g1, be1, m1, v1 = p_ref[1:2, :], p_ref[2:3, :], p_ref[3:4, :], p_ref[4:5, :]
    b2 = p_ref[5:6, :]
    g2, be2, m2, v2 = p_ref[6:7, :], p_ref[7:8, :], p_ref[8:9, :], p_ref[9:10, :]
    o = jnp.concatenate([olo_ref[...], ohi_ref[...]], axis=1)
    y = x_ref[...] + o + gat_bias
    y = (y - m1) * lax.rsqrt(v1 + EPS) * g1 + be1
    hid = jnp.maximum(jnp.dot(y, w1_ref[...], preferred_element_type=f32)
                      + b1_ref[...], 0.0)
    ff = jnp.dot(hid, w2_ref[...], preferred_element_type=f32) + b2
    z = y + ff
    out_ref[...] = (z - m2) * lax.rsqrt(v2 + EPS) * g2 + be2


def _post(x, olo, ohi, params, W1, b1, W2):
    blk = 1000
    grid = (N // blk,)
    return pl.pallas_call(
        _post_body,
        grid=grid,
        in_specs=[
            pl.BlockSpec((blk, D), lambda i: (i, 0)),
            pl.BlockSpec((blk, D // 2), lambda i: (i, 0)),
            pl.BlockSpec((blk, D // 2), lambda i: (i, 0)),
            pl.BlockSpec((10, D), lambda i: (0, 0)),
            pl.BlockSpec((D, D_INNER), lambda i: (0, 0)),
            pl.BlockSpec((1, D_INNER), lambda i: (0, 0)),
            pl.BlockSpec((D_INNER, D), lambda i: (0, 0)),
        ],
        out_specs=pl.BlockSpec((blk, D), lambda i: (i, 0)),
        out_shape=jax.ShapeDtypeStruct((N, D), f32),
    )(x, olo, ohi, params, W1, b1.reshape(1, D_INNER), W2)


def kernel(x, edge_index, W_l, b_l, W_r, b_r, att, gat_bias,
           bn1_gamma, bn1_beta, bn1_mean, bn1_var,
           W1, b1, W2, b2,
           bn2_gamma, bn2_beta, bn2_mean, bn2_var):
    loop = jnp.arange(N, dtype=i32)
    pad = EP - EDGES
    src = jnp.concatenate([edge_index[0].astype(i32), loop,
                           jnp.zeros((pad,), i32)])
    dst = jnp.concatenate([edge_index[1].astype(i32), loop,
                           jnp.full((pad,), N, i32)])

    row = lax.shift_right_logical(dst, 3)
    orow = lax.shift_right_logical(dst, 1)
    src2 = jnp.concatenate([src, src + N])

    xl, xr = _pre(x, W_l, b_l, W_r, b_r)

    def pack(a):
        m, w = a.shape
        t = a.reshape(m, w // 32, 2, 16).transpose(0, 1, 3, 2).reshape(m, w)
        return lax.bitcast_convert_type(
            t.astype(jnp.bfloat16).reshape(m, w // 2, 2), i32)

    xl_pk = pack(xl)
    xr_pk = pack(xr)
    ex, denom = _passA(src, dst, row, xl_pk, xr_pk, att)
    d0, d1 = denom[0], denom[1]
    xl3 = xl.reshape(N, H, C)
    xlh = jnp.concatenate([xl3[:, :, :C // 2].reshape(N, HC // 2),
                           xl3[:, :, C // 2:].reshape(N, HC // 2)], axis=0)
    out2 = _passB(src2, dst, row, orow, ex, pack(xlh), d0, d1)
    olo = out2[0].reshape(NP, C // 2)[:N]
    ohi = out2[1].reshape(NP, C // 2)[:N]

    params = jnp.stack([gat_bias, bn1_gamma, bn1_beta, bn1_mean, bn1_var,
                        b2, bn2_gamma, bn2_beta, bn2_mean, bn2_var])
    return _post(x, olo, ohi, params, W1, b1, W2)


# speedup vs baseline: 1.5665x; 1.0079x over previous
import functools

import jax
import jax.numpy as jnp
from jax import lax
from jax.experimental import pallas as pl
from jax.experimental.pallas import tpu as pltpu
from jax.experimental.pallas import tpu_sc as plsc

N = 10000
E = 320000
D = 128
H = 8
C = 128
HC = H * C
D_INNER = 512
EPS = 1e-5

NC = 2
NS = 16
NW = NC * NS

EDGES = E + N
EP = 330240
SPAN = EP // NW
B = 40
ITERS = SPAN // B
BB = 40
SPAN_B = EP // NS
ITERS_B = SPAN_B // BB
NP = 10240
ZR = NP // NS
PR = NP // 8
PB = NP // 2
ZO = PB // NS

f32 = jnp.float32
i32 = jnp.int32


def _pre_body(x_ref, wl_ref, bl_ref, wr_ref, br_ref, xl_ref, xr_ref):
    x = x_ref[...]
    xl_ref[...] = jnp.dot(x, wl_ref[...], preferred_element_type=f32) + bl_ref[...]
    xr_ref[...] = jnp.dot(x, wr_ref[...], preferred_element_type=f32) + br_ref[...]


def _pre(x, W_l, b_l, W_r, b_r):
    blk = 1000
    grid = (N // blk,)
    return pl.pallas_call(
        _pre_body,
        grid=grid,
        in_specs=[
            pl.BlockSpec((blk, D), lambda i: (i, 0)),
            pl.BlockSpec((D, HC), lambda i: (0, 0)),
            pl.BlockSpec((1, HC), lambda i: (0, 0)),
            pl.BlockSpec((D, HC), lambda i: (0, 0)),
            pl.BlockSpec((1, HC), lambda i: (0, 0)),
        ],
        out_specs=[
            pl.BlockSpec((blk, HC), lambda i: (i, 0)),
            pl.BlockSpec((blk, HC), lambda i: (i, 0)),
        ],
        out_shape=[
            jax.ShapeDtypeStruct((N, HC), f32),
            jax.ShapeDtypeStruct((N, HC), f32),
        ],
    )(x, W_l, b_l.reshape(1, HC), W_r, b_r.reshape(1, HC))


def _passA_body(src_hbm, dst_hbm, row_hbm, xl_hbm, xr_hbm, att_hbm,
                ex_hbm, denom_hbm,
                idx_s, idx_d, rowidx, rows_l, rows_r, exd_v, exs_v, att_v,
                zbuf, denom_sh, sem, semg1, semg2):
    c = lax.axis_index("c")
    s = lax.axis_index("s")
    wid = c * NS + s
    lane = lax.iota(i32, 16)
    zrow = jnp.zeros((16,), f32)

    for k in range(8 * (C // 16)):
        zbuf[k // 8, pl.ds((k % 8) * 16, 16)] = zrow

    @pl.when(s < 2)
    def _():
        def zcopy(r, _):
            pltpu.sync_copy(zbuf, denom_sh.at[pl.ds(s * (PR // 2) + r * 8, 8)])
            return 0
        lax.fori_loop(0, PR // 16, zcopy, 0)

    pltpu.sync_copy(att_hbm, att_v)
    plsc.subcore_barrier()

    base = wid * SPAN

    def step(it, _):
        off = base + it * B
        m1 = pltpu.async_copy(src_hbm.at[pl.ds(off, B)], idx_s, sem)
        m2 = pltpu.async_copy(dst_hbm.at[pl.ds(off, B)], idx_d, sem)
        m3 = pltpu.async_copy(row_hbm.at[pl.ds(off, B)], rowidx, sem)
        m1.wait(); m2.wait(); m3.wait()
        g1 = pltpu.async_copy(xl_hbm.at[idx_s], rows_l, semg1)
        g2 = pltpu.async_copy(xr_hbm.at[idx_d], rows_r, semg2)
        g1.wait(); g2.wait()

        hmask = jnp.full((16,), -0x10000, i32)

        def edge(e, _):
            svec = jnp.full((16,), -1e30, f32)
            for h in range(H):
                acc = jnp.zeros((16,), f32)
                for g in range(C // 32):
                    o = h * (C // 2) + g * 16
                    ul = rows_l[e, pl.ds(o, 16)]
                    ur = rows_r[e, pl.ds(o, 16)]
                    lo = (plsc.bitcast(lax.shift_left(ul, 16), f32)
                          + plsc.bitcast(lax.shift_left(ur, 16), f32))
                    hi = (plsc.bitcast(ul & hmask, f32)
                          + plsc.bitcast(ur & hmask, f32))
                    llo = jnp.maximum(lo, lo * 0.2)
                    lhi = jnp.maximum(hi, hi * 0.2)
                    acc = acc + llo * att_v[h, pl.ds(g * 32, 16)]
                    acc = acc + lhi * att_v[h, pl.ds(g * 32 + 16, 16)]
                svec = jnp.where(lane == h, jnp.sum(acc), svec)
            ex = jnp.exp(svec)
            exs_v[e, :] = ex
            for k in range(C // 16):
                exd_v[e, pl.ds(k * 16, 16)] = zrow
            ebc = jnp.zeros((16,), i32) + e
            dbc = plsc.load_gather(idx_d, [ebc])
            col = (dbc & 7) * 16 + lane
            plsc.store_scatter(exd_v, [ebc, col], ex)
            return 0

        lax.fori_loop(0, B, edge, 0)
        pltpu.sync_copy(exd_v, denom_sh.at[rowidx], add=True)
        pltpu.sync_copy(exs_v, ex_hbm.at[pl.ds(off, B)])
        return 0

    lax.fori_loop(0, ITERS, step, 0)
    plsc.subcore_barrier()

    @pl.when(s < 2)
    def _():
        pltpu.sync_copy(denom_sh.at[pl.ds(s * (PR // 2), PR // 2)],
                        denom_hbm.at[c, pl.ds(s * (PR // 2), PR // 2)])


def _passA(src, dst, row, xl, xr, att):
    mesh = plsc.VectorSubcoreMesh(core_axis_name="c", subcore_axis_name="s")
    kern = pl.kernel(
        _passA_body,
        compiler_params=pltpu.CompilerParams(needs_layout_passes=False),
        out_type=[
            jax.ShapeDtypeStruct((EP, 16), f32),
            jax.ShapeDtypeStruct((NC, PR, C), f32),
        ],
        mesh=mesh,
        scratch_types=[
            pltpu.VMEM((B,), i32),
            pltpu.VMEM((B,), i32),
            pltpu.VMEM((B,), i32),
            pltpu.VMEM((B, HC // 2), i32),
            pltpu.VMEM((B, HC // 2), i32),
            pltpu.VMEM((B, C), f32),
            pltpu.VMEM((B, 16), f32),
            pltpu.VMEM((H, C), f32),
            pltpu.VMEM((8, C), f32),
            pltpu.VMEM_SHARED((PR, C), f32),
            pltpu.SemaphoreType.DMA,
            pltpu.SemaphoreType.DMA,
            pltpu.SemaphoreType.DMA,
        ],
    )
    return kern(src, dst, row, xl, xr, att)


def _passB_body(src2_hbm, dst_hbm, row_hbm, orow_hbm, ex_hbm, xlh_hbm,
                d0_hbm, d1_hbm,
                out_hbm,
                idx_s, idx_d, rowidx, orowidx, rows_l, ex_v, d0_v, d1_v,
                contrib, zbuf, out_sh, sem, semg1, semg2, semg3):
    c = lax.axis_index("c")
    s = lax.axis_index("s")
    wid = c * NS + s
    lane = lax.iota(i32, 16)

    zrow = jnp.zeros((16,), f32)
    for k in range(8 * (C // 16)):
        zbuf[k // 8, pl.ds((k % 8) * 16, 16)] = zrow

    def zcopy(r, _):
        pltpu.sync_copy(zbuf, out_sh.at[pl.ds(s * ZO + r * 8, 8)])
        return 0
    lax.fori_loop(0, ZO // 8, zcopy, 0)
    plsc.subcore_barrier()

    base = s * SPAN_B

    def step(it, _):
        off = base + it * BB
        m1 = pltpu.async_copy(src2_hbm.at[pl.ds(c * EP + off, BB)], idx_s, sem)
        m2 = pltpu.async_copy(dst_hbm.at[pl.ds(off, BB)], idx_d, sem)
        m3 = pltpu.async_copy(row_hbm.at[pl.ds(off, BB)], rowidx, sem)
        m5 = pltpu.async_copy(orow_hbm.at[pl.ds(off, BB)], orowidx, sem)
        m4 = pltpu.async_copy(ex_hbm.at[pl.ds(off, BB)], ex_v, sem)
        m1.wait(); m2.wait(); m3.wait(); m4.wait(); m5.wait()
        g1 = pltpu.async_copy(xlh_hbm.at[idx_s], rows_l, semg1)
        g2 = pltpu.async_copy(d0_hbm.at[rowidx], d0_v, semg2)
        g3 = pltpu.async_copy(d1_hbm.at[rowidx], d1_v, semg3)
        g1.wait(); g2.wait(); g3.wait()

        def edge(e, _):
            ebc = jnp.zeros((16,), i32) + e
            dbc = plsc.load_gather(idx_d, [ebc])
            col = (dbc & 7) * 16 + lane
            d = (plsc.load_gather(d0_v, [ebc, col])
                 + plsc.load_gather(d1_v, [ebc, col]))
            alpha = ex_v[e, :] / (d + 1e-16) * (1.0 / H)
            ah = [alpha.at[jnp.full((16,), h, i32)].get(mode="promise_in_bounds")
                  for h in range(H)]
            cb = (dbc & 1) * 64 + lane
            ncb = (1 - (dbc & 1)) * 64 + lane
            hmask = jnp.full((16,), -0x10000, i32)
            for g in range(2):
                acc_lo = jnp.zeros((16,), f32)
                acc_hi = jnp.zeros((16,), f32)
                for h in range(H):
                    u = rows_l[e, pl.ds(h * 32 + g * 16, 16)]
                    acc_lo = acc_lo + ah[h] * plsc.bitcast(
                        lax.shift_left(u, 16), f32)
                    acc_hi = acc_hi + ah[h] * plsc.bitcast(u & hmask, f32)
                plsc.store_scatter(contrib, [ebc, cb + g * 32], acc_lo)
                plsc.store_scatter(contrib, [ebc, cb + g * 32 + 16], acc_hi)
                plsc.store_scatter(contrib, [ebc, ncb + g * 32], zrow)
                plsc.store_scatter(contrib, [ebc, ncb + g * 32 + 16], zrow)
            return 0

        lax.fori_loop(0, BB, edge, 0)
        pltpu.sync_copy(contrib, out_sh.at[orowidx], add=True)
        return 0

    lax.fori_loop(0, ITERS_B, step, 0)
    plsc.subcore_barrier()
    pltpu.sync_copy(out_sh.at[pl.ds(s * ZO, ZO)],
                    out_hbm.at[c, pl.ds(s * ZO, ZO)])


def _passB(src2, dst, row, orow, ex, xlh, d0, d1):
    mesh = plsc.VectorSubcoreMesh(core_axis_name="c", subcore_axis_name="s")
    kern = pl.kernel(
        _passB_body,
        compiler_params=pltpu.CompilerParams(needs_layout_passes=False),
        out_type=jax.ShapeDtypeStruct((NC, PB, C), f32),
        mesh=mesh,
        scratch_types=[
            pltpu.VMEM((BB,), i32),
            pltpu.VMEM((BB,), i32),
            pltpu.VMEM((BB,), i32),
            pltpu.VMEM((BB,), i32),
            pltpu.VMEM((BB, HC // 4), i32),
            pltpu.VMEM((BB, 16), f32),
            pltpu.VMEM((BB, C), f32),
            pltpu.VMEM((BB, C), f32),
            pltpu.VMEM((BB, C), f32),
            pltpu.VMEM((8, C), f32),
            pltpu.VMEM_SHARED((PB, C), f32),
            pltpu.SemaphoreType.DMA,
            pltpu.SemaphoreType.DMA,
            pltpu.SemaphoreType.DMA,
            pltpu.SemaphoreType.DMA,
        ],
    )
    return kern(src2, dst, row, orow, ex, xlh, d0, d1)


def _post_body(x_ref, olo_ref, ohi_ref, p_ref, w1_ref, b1_ref, w2_ref, out_ref):
    gat_bias = p_ref[0:1, :]
    g1, be1, m1, v1 = p_ref[1:2, :], p_ref[2:3, :], p_ref[3:4, :], p_ref[4:5, :]
    b2 = p_ref[5:6, :]
    g2, be2, m2, v2 = p_ref[6:7, :], p_ref[7:8, :], p_ref[8:9, :], p_ref[9:10, :]
    o = jnp.concatenate([olo_ref[...], ohi_ref[...]], axis=1)
    y = x_ref[...] + o + gat_bias
    y = (y - m1) * lax.rsqrt(v1 + EPS) * g1 + be1
    hid = jnp.maximum(jnp.dot(y, w1_ref[...], preferred_element_type=f32)
                      + b1_ref[...], 0.0)
    ff = jnp.dot(hid, w2_ref[...], preferred_element_type=f32) + b2
    z = y + ff
    out_ref[...] = (z - m2) * lax.rsqrt(v2 + EPS) * g2 + be2


def _post(x, olo, ohi, params, W1, b1, W2):
    blk = 1000
    grid = (N // blk,)
    return pl.pallas_call(
        _post_body,
        grid=grid,
        in_specs=[
            pl.BlockSpec((blk, D), lambda i: (i, 0)),
            pl.BlockSpec((blk, D // 2), lambda i: (i, 0)),
            pl.BlockSpec((blk, D // 2), lambda i: (i, 0)),
            pl.BlockSpec((10, D), lambda i: (0, 0)),
            pl.BlockSpec((D, D_INNER), lambda i: (0, 0)),
            pl.BlockSpec((1, D_INNER), lambda i: (0, 0)),
            pl.BlockSpec((D_INNER, D), lambda i: (0, 0)),
        ],
        out_specs=pl.BlockSpec((blk, D), lambda i: (i, 0)),
        out_shape=jax.ShapeDtypeStruct((N, D), f32),
    )(x, olo, ohi, params, W1, b1.reshape(1, D_INNER), W2)


def kernel(x, edge_index, W_l, b_l, W_r, b_r, att, gat_bias,
           bn1_gamma, bn1_beta, bn1_mean, bn1_var,
           W1, b1, W2, b2,
           bn2_gamma, bn2_beta, bn2_mean, bn2_var):
    loop = jnp.arange(N, dtype=i32)
    pad = EP - EDGES
    src = jnp.concatenate([edge_index[0].astype(i32), loop,
                           jnp.zeros((pad,), i32)])
    dst = jnp.concatenate([edge_index[1].astype(i32), loop,
                           jnp.full((pad,), N, i32)])

    row = lax.shift_right_logical(dst, 3)
    orow = lax.shift_right_logical(dst, 1)
    src2 = jnp.concatenate([src, src + N])

    xl, xr = _pre(x, W_l, b_l, W_r, b_r)

    def pack(a):
        m, w = a.shape
        t = a.reshape(m, w // 32, 2, 16).transpose(0, 1, 3, 2).reshape(m, w)
        return lax.bitcast_convert_type(
            t.astype(jnp.bfloat16).reshape(m, w // 2, 2), i32)

    xl_pk = pack(xl)
    xr_pk = pack(xr)
    ex, denom = _passA(src, dst, row, xl_pk, xr_pk, att)
    d0, d1 = denom[0], denom[1]
    xl3 = xl.reshape(N, H, C)
    xlh = jnp.concatenate([xl3[:, :, :C // 2].reshape(N, HC // 2),
                           xl3[:, :, C // 2:].reshape(N, HC // 2)], axis=0)
    out2 = _passB(src2, dst, row, orow, ex, pack(xlh), d0, d1)
    olo = out2[0].reshape(NP, C // 2)[:N]
    ohi = out2[1].reshape(NP, C // 2)[:N]

    params = jnp.stack([gat_bias, bn1_gamma, bn1_beta, bn1_mean, bn1_var,
                        b2, bn2_gamma, bn2_beta, bn2_mean, bn2_var])
    return _post(x, olo, ohi, params, W1, b1, W2)


# speedup vs baseline: 1.6861x; 1.0763x over previous
import functools

import jax
import jax.numpy as jnp
from jax import lax
from jax.experimental import pallas as pl
from jax.experimental.pallas import tpu as pltpu
from jax.experimental.pallas import tpu_sc as plsc

N = 10000
E = 320000
D = 128
H = 8
C = 128
HC = H * C
D_INNER = 512
EPS = 1e-5

NC = 2
NS = 16
NW = NC * NS

EDGES = E + N
EP = 330240
SPAN = EP // NW
B = 40
ITERS = SPAN // B
BB = 48
SPAN_B = EP // NS
ITERS_B = SPAN_B // BB
NP = 10240
ZR = NP // NS
PR = NP // 8
PB = NP // 2
ZO = PB // NS

f32 = jnp.float32
i32 = jnp.int32


def _pre_body(x_ref, wl_ref, bl_ref, wr_ref, br_ref, xl_ref, xr_ref):
    x = x_ref[...]
    xl_ref[...] = jnp.dot(x, wl_ref[...], preferred_element_type=f32) + bl_ref[...]
    xr_ref[...] = jnp.dot(x, wr_ref[...], preferred_element_type=f32) + br_ref[...]


def _pre(x, W_l, b_l, W_r, b_r):
    blk = 1000
    grid = (N // blk,)
    return pl.pallas_call(
        _pre_body,
        grid=grid,
        in_specs=[
            pl.BlockSpec((blk, D), lambda i: (i, 0)),
            pl.BlockSpec((D, HC), lambda i: (0, 0)),
            pl.BlockSpec((1, HC), lambda i: (0, 0)),
            pl.BlockSpec((D, HC), lambda i: (0, 0)),
            pl.BlockSpec((1, HC), lambda i: (0, 0)),
        ],
        out_specs=[
            pl.BlockSpec((blk, HC), lambda i: (i, 0)),
            pl.BlockSpec((blk, HC), lambda i: (i, 0)),
        ],
        out_shape=[
            jax.ShapeDtypeStruct((N, HC), f32),
            jax.ShapeDtypeStruct((N, HC), f32),
        ],
    )(x, W_l, b_l.reshape(1, HC), W_r, b_r.reshape(1, HC))


def _passA_body(src_hbm, dst_hbm, row_hbm, xl_hbm, xr_hbm, att_hbm,
                ex_hbm, denom_hbm,
                idx_s, idx_d, rowidx, rows_l, rows_r, exd_v, exs_v, att_v,
                zbuf, denom_sh, sem, semg1, semg2):
    c = lax.axis_index("c")
    s = lax.axis_index("s")
    wid = c * NS + s
    lane = lax.iota(i32, 16)
    zrow = jnp.zeros((16,), f32)

    for k in range(8 * (C // 16)):
        zbuf[k // 8, pl.ds((k % 8) * 16, 16)] = zrow

    @pl.when(s < 2)
    def _():
        def zcopy(r, _):
            pltpu.sync_copy(zbuf, denom_sh.at[pl.ds(s * (PR // 2) + r * 8, 8)])
            return 0
        lax.fori_loop(0, PR // 16, zcopy, 0)

    pltpu.sync_copy(att_hbm, att_v)
    plsc.subcore_barrier()

    base = wid * SPAN

    def step(it, _):
        off = base + it * B
        m1 = pltpu.async_copy(src_hbm.at[pl.ds(off, B)], idx_s, sem)
        m2 = pltpu.async_copy(dst_hbm.at[pl.ds(off, B)], idx_d, sem)
        m3 = pltpu.async_copy(row_hbm.at[pl.ds(off, B)], rowidx, sem)
        m1.wait(); m2.wait(); m3.wait()
        g1 = pltpu.async_copy(xl_hbm.at[idx_s], rows_l, semg1)
        g2 = pltpu.async_copy(xr_hbm.at[idx_d], rows_r, semg2)
        g1.wait(); g2.wait()

        hmask = jnp.full((16,), -0x10000, i32)

        def edge(e, _):
            svec = jnp.full((16,), -1e30, f32)
            for h in range(H):
                acc = jnp.zeros((16,), f32)
                for g in range(C // 32):
                    o = h * (C // 2) + g * 16
                    ul = rows_l[e, pl.ds(o, 16)]
                    ur = rows_r[e, pl.ds(o, 16)]
                    lo = (plsc.bitcast(lax.shift_left(ul, 16), f32)
                          + plsc.bitcast(lax.shift_left(ur, 16), f32))
                    hi = (plsc.bitcast(ul & hmask, f32)
                          + plsc.bitcast(ur & hmask, f32))
                    llo = jnp.maximum(lo, lo * 0.2)
                    lhi = jnp.maximum(hi, hi * 0.2)
                    acc = acc + llo * att_v[h, pl.ds(g * 32, 16)]
                    acc = acc + lhi * att_v[h, pl.ds(g * 32 + 16, 16)]
                svec = jnp.where(lane == h, jnp.sum(acc), svec)
            ex = jnp.exp(svec)
            exs_v[e, :] = ex
            for k in range(C // 16):
                exd_v[e, pl.ds(k * 16, 16)] = zrow
            ebc = jnp.zeros((16,), i32) + e
            dbc = plsc.load_gather(idx_d, [ebc])
            col = (dbc & 7) * 16 + lane
            plsc.store_scatter(exd_v, [ebc, col], ex)
            return 0

        lax.fori_loop(0, B, edge, 0)
        pltpu.sync_copy(exd_v, denom_sh.at[rowidx], add=True)
        pltpu.sync_copy(exs_v, ex_hbm.at[pl.ds(off, B)])
        return 0

    lax.fori_loop(0, ITERS, step, 0)
    plsc.subcore_barrier()

    @pl.when(s < 2)
    def _():
        pltpu.sync_copy(denom_sh.at[pl.ds(s * (PR // 2), PR // 2)],
                        denom_hbm.at[c, pl.ds(s * (PR // 2), PR // 2)])


def _passA(src, dst, row, xl, xr, att):
    mesh = plsc.VectorSubcoreMesh(core_axis_name="c", subcore_axis_name="s")
    kern = pl.kernel(
        _passA_body,
        compiler_params=pltpu.CompilerParams(needs_layout_passes=False),
        out_type=[
            jax.ShapeDtypeStruct((EP, 16), f32),
            jax.ShapeDtypeStruct((NC, PR, C), f32),
        ],
        mesh=mesh,
        scratch_types=[
            pltpu.VMEM((B,), i32),
            pltpu.VMEM((B,), i32),
            pltpu.VMEM((B,), i32),
            pltpu.VMEM((B, HC // 2), i32),
            pltpu.VMEM((B, HC // 2), i32),
            pltpu.VMEM((B, C), f32),
            pltpu.VMEM((B, 16), f32),
            pltpu.VMEM((H, C), f32),
            pltpu.VMEM((8, C), f32),
            pltpu.VMEM_SHARED((PR, C), f32),
            pltpu.SemaphoreType.DMA,
            pltpu.SemaphoreType.DMA,
            pltpu.SemaphoreType.DMA,
        ],
    )
    return kern(src, dst, row, xl, xr, att)


def _passB_body(src2_hbm, dst_hbm, row_hbm, orow_hbm, ex_hbm, xlh_hbm,
                d_hbm,
                out_hbm,
                idx_s, idx_d, rowidx, orowidx, rows_l, ex_v, d_v,
                contrib, zbuf, out_sh, sem, semg1, semg2):
    c = lax.axis_index("c")
    s = lax.axis_index("s")
    wid = c * NS + s
    lane = lax.iota(i32, 16)

    zrow = jnp.zeros((16,), f32)
    for k in range(8 * (C // 16)):
        zbuf[k // 8, pl.ds((k % 8) * 16, 16)] = zrow

    def zcopy(r, _):
        pltpu.sync_copy(zbuf, out_sh.at[pl.ds(s * ZO + r * 8, 8)])
        return 0
    lax.fori_loop(0, ZO // 8, zcopy, 0)
    plsc.subcore_barrier()

    base = s * SPAN_B

    def step(it, _):
        off = base + it * BB
        m1 = pltpu.async_copy(src2_hbm.at[pl.ds(c * EP + off, BB)], idx_s, sem)
        m2 = pltpu.async_copy(dst_hbm.at[pl.ds(off, BB)], idx_d, sem)
        m3 = pltpu.async_copy(row_hbm.at[pl.ds(off, BB)], rowidx, sem)
        m5 = pltpu.async_copy(orow_hbm.at[pl.ds(off, BB)], orowidx, sem)
        m4 = pltpu.async_copy(ex_hbm.at[pl.ds(off, BB)], ex_v, sem)
        m1.wait(); m2.wait(); m3.wait(); m4.wait(); m5.wait()
        g1 = pltpu.async_copy(xlh_hbm.at[idx_s], rows_l, semg1)
        g2 = pltpu.async_copy(d_hbm.at[rowidx], d_v, semg2)
        g1.wait(); g2.wait()

        def edge(e, _):
            ebc = jnp.zeros((16,), i32) + e
            dbc = plsc.load_gather(idx_d, [ebc])
            col = (dbc & 7) * 16 + lane
            d = plsc.load_gather(d_v, [ebc, col])
            alpha = ex_v[e, :] / (d + 1e-16) * (1.0 / H)
            ah = [alpha.at[jnp.full((16,), h, i32)].get(mode="promise_in_bounds")
                  for h in range(H)]
            cb = (dbc & 1) * 64 + lane
            ncb = (1 - (dbc & 1)) * 64 + lane
            hmask = jnp.full((16,), -0x10000, i32)
            for g in range(2):
                acc_lo = jnp.zeros((16,), f32)
                acc_hi = jnp.zeros((16,), f32)
                for h in range(H):
                    u = rows_l[e, pl.ds(h * 32 + g * 16, 16)]
                    acc_lo = acc_lo + ah[h] * plsc.bitcast(
                        lax.shift_left(u, 16), f32)
                    acc_hi = acc_hi + ah[h] * plsc.bitcast(u & hmask, f32)
                plsc.store_scatter(contrib, [ebc, cb + g * 32], acc_lo)
                plsc.store_scatter(contrib, [ebc, cb + g * 32 + 16], acc_hi)
                plsc.store_scatter(contrib, [ebc, ncb + g * 32], zrow)
                plsc.store_scatter(contrib, [ebc, ncb + g * 32 + 16], zrow)
            return 0

        lax.fori_loop(0, BB, edge, 0)
        pltpu.sync_copy(contrib, out_sh.at[orowidx], add=True)
        return 0

    lax.fori_loop(0, ITERS_B, step, 0)
    plsc.subcore_barrier()
    pltpu.sync_copy(out_sh.at[pl.ds(s * ZO, ZO)],
                    out_hbm.at[c, pl.ds(s * ZO, ZO)])


def _passB(src2, dst, row, orow, ex, xlh, d):
    mesh = plsc.VectorSubcoreMesh(core_axis_name="c", subcore_axis_name="s")
    kern = pl.kernel(
        _passB_body,
        compiler_params=pltpu.CompilerParams(needs_layout_passes=False),
        out_type=jax.ShapeDtypeStruct((NC, PB, C), f32),
        mesh=mesh,
        scratch_types=[
            pltpu.VMEM((BB,), i32),
            pltpu.VMEM((BB,), i32),
            pltpu.VMEM((BB,), i32),
            pltpu.VMEM((BB,), i32),
            pltpu.VMEM((BB, HC // 4), i32),
            pltpu.VMEM((BB, 16), f32),
            pltpu.VMEM((BB, C), f32),
            pltpu.VMEM((BB, C), f32),
            pltpu.VMEM((8, C), f32),
            pltpu.VMEM_SHARED((PB, C), f32),
            pltpu.SemaphoreType.DMA,
            pltpu.SemaphoreType.DMA,
            pltpu.SemaphoreType.DMA,
        ],
    )
    return kern(src2, dst, row, orow, ex, xlh, d)


def _post_body(x_ref, olo_ref, ohi_ref, p_ref, w1_ref, b1_ref, w2_ref, out_ref):
    gat_bias = p_ref[0:1, :]
    g1, be1, m1, v1 = p_ref[1:2, :], p_ref[2:3, :], p_ref[3:4, :], p_ref[4:5, :]
    b2 = p_ref[5:6, :]
    g2, be2, m2, v2 = p_ref[6:7, :], p_ref[7:8, :], p_ref[8:9, :], p_ref[9:10, :]
    o = jnp.concatenate([olo_ref[...], ohi_ref[...]], axis=1)
    y = x_ref[...] + o + gat_bias
    y = (y - m1) * lax.rsqrt(v1 + EPS) * g1 + be1
    hid = jnp.maximum(jnp.dot(y, w1_ref[...], preferred_element_type=f32)
                      + b1_ref[...], 0.0)
    ff = jnp.dot(hid, w2_ref[...], preferred_element_type=f32) + b2
    z = y + ff
    out_ref[...] = (z - m2) * lax.rsqrt(v2 + EPS) * g2 + be2


def _post(x, olo, ohi, params, W1, b1, W2):
    blk = 1000
    grid = (N // blk,)
    return pl.pallas_call(
        _post_body,
        grid=grid,
        in_specs=[
            pl.BlockSpec((blk, D), lambda i: (i, 0)),
            pl.BlockSpec((blk, D // 2), lambda i: (i, 0)),
            pl.BlockSpec((blk, D // 2), lambda i: (i, 0)),
            pl.BlockSpec((10, D), lambda i: (0, 0)),
            pl.BlockSpec((D, D_INNER), lambda i: (0, 0)),
            pl.BlockSpec((1, D_INNER), lambda i: (0, 0)),
            pl.BlockSpec((D_INNER, D), lambda i: (0, 0)),
        ],
        out_specs=pl.BlockSpec((blk, D), lambda i: (i, 0)),
        out_shape=jax.ShapeDtypeStruct((N, D), f32),
    )(x, olo, ohi, params, W1, b1.reshape(1, D_INNER), W2)


def kernel(x, edge_index, W_l, b_l, W_r, b_r, att, gat_bias,
           bn1_gamma, bn1_beta, bn1_mean, bn1_var,
           W1, b1, W2, b2,
           bn2_gamma, bn2_beta, bn2_mean, bn2_var):
    loop = jnp.arange(N, dtype=i32)
    pad = EP - EDGES
    src = jnp.concatenate([edge_index[0].astype(i32), loop,
                           jnp.zeros((pad,), i32)])
    dst = jnp.concatenate([edge_index[1].astype(i32), loop,
                           jnp.full((pad,), N, i32)])

    row = lax.shift_right_logical(dst, 3)
    orow = lax.shift_right_logical(dst, 1)
    src2 = jnp.concatenate([src, src + N])

    xl, xr = _pre(x, W_l, b_l, W_r, b_r)

    def pack(a):
        m, w = a.shape
        t = a.reshape(m, w // 32, 2, 16).transpose(0, 1, 3, 2).reshape(m, w)
        return lax.bitcast_convert_type(
            t.astype(jnp.bfloat16).reshape(m, w // 2, 2), i32)

    xl_pk = pack(xl)
    xr_pk = pack(xr)
    ex, denom = _passA(src, dst, row, xl_pk, xr_pk, att)
    dsum = denom[0] + denom[1]
    xl3 = xl.reshape(N, H, C)
    xlh = jnp.concatenate([xl3[:, :, :C // 2].reshape(N, HC // 2),
                           xl3[:, :, C // 2:].reshape(N, HC // 2)], axis=0)
    out2 = _passB(src2, dst, row, orow, ex, pack(xlh), dsum)
    olo = out2[0].reshape(NP, C // 2)[:N]
    ohi = out2[1].reshape(NP, C // 2)[:N]

    params = jnp.stack([gat_bias, bn1_gamma, bn1_beta, bn1_mean, bn1_var,
                        b2, bn2_gamma, bn2_beta, bn2_mean, bn2_var])
    return _post(x, olo, ohi, params, W1, b1, W2)
